# baseline (speedup 1.0000x reference)
import jax
import jax.numpy as jnp
from jax.experimental import pallas as pl
from jax.experimental.pallas import tpu as pltpu

_PENALTY = 0.1


def _controls_body(w_ref, b_ref, obs_ref, out_ref, vbuf, sem):
    b = pl.program_id(0)
    t = pl.program_id(1)
    n_b = pl.num_programs(0)
    n_t = pl.num_programs(1)
    step = b * n_t + t
    slot = step % 2

    blk = obs_ref.shape[1]
    ts = obs_ref.shape[2]
    c = obs_ref.shape[3]
    n = blk * ts
    m = obs_ref[0].reshape(n, c)
    x = jax.lax.dot_general(
        w_ref[...], m, (((1,), (1,)), ((), ())),
        preferred_element_type=jnp.float32,
    ) + b_ref[0, 0]
    s = jax.nn.log_sigmoid(x)
    il = jnp.stack([s, s - x], axis=-1).reshape(1, 2 * n)

    full = 2 * n
    last = full - 2 * ts
    row0 = t * full

    def _copy(sl, off, size):
        return pltpu.make_async_copy(
            vbuf.at[sl, :, pl.ds(0, size)],
            out_ref.at[b, :, pl.ds(off, size)],
            sem.at[sl],
        )

    @pl.when((step >= 2) & (t != 1))
    def _():
        _copy(slot, row0, full).wait()

    @pl.when((step >= 2) & (t == 1))
    def _():
        _copy(slot, row0, last).wait()

    vbuf[slot] = il

    @pl.when(t < n_t - 1)
    def _():
        _copy(slot, row0, full).start()

    @pl.when(t == n_t - 1)
    def _():
        _copy(slot, row0, last).start()

    @pl.when((b == n_b - 1) & (t == n_t - 1))
    def _():
        _copy(slot, row0, last).wait()
        _copy(1 - slot, row0 - full, full).wait()


def _gamma_body(scores_ref, gamma_ref, read_ref, write_ref):
    sc = scores_ref[0]
    Tt, Ts = sc.shape
    lane_i = jax.lax.broadcasted_iota(jnp.int32, (Tt, Ts), 1)
    lane_f = lane_i.astype(jnp.float32)
    scp = sc - _PENALTY * (lane_f / Ts)
    m = jnp.max(scp, axis=1, keepdims=True)
    cand = jnp.where(scp == m, lane_i, Ts)
    bc = jnp.min(cand, axis=1, keepdims=True)
    sub_i = jax.lax.broadcasted_iota(jnp.int32, (Tt, 1), 0)
    k = 1
    while k < Tt:
        shifted = pltpu.roll(bc, k, axis=0)
        bc = jnp.maximum(bc, jnp.where(sub_i >= k, shifted, -1))
        k *= 2
    gamma = (lane_i >= bc).astype(jnp.float32)
    gamma_ref[0] = gamma
    write_ref[0] = gamma[1:, :]
    read_ref[0] = 1.0 - gamma[1:, :]


def _run(observations, scores, gate_w, gate_b, interpret=False):
    B, Tt, Ts, C = observations.shape
    Tblk = 32
    nT = Tt // Tblk
    controls_flat = pl.pallas_call(
        _controls_body,
        grid=(B, nT),
        in_specs=[
            pl.BlockSpec((1, C), lambda b, t: (0, 0)),
            pl.BlockSpec((1, 1), lambda b, t: (0, 0)),
            pl.BlockSpec((1, Tblk, Ts, C), lambda b, t: (b, t, 0, 0)),
        ],
        out_specs=pl.BlockSpec(memory_space=pl.ANY),
        out_shape=jax.ShapeDtypeStruct((B, 1, (Tt - 1) * Ts * 2), jnp.float32),
        scratch_shapes=[
            pltpu.VMEM((2, 1, Tblk * Ts * 2), jnp.float32),
            pltpu.SemaphoreType.DMA((2,)),
        ],
        interpret=interpret,
    )(gate_w, gate_b.reshape(1, 1), observations)
    controls = controls_flat.reshape(B, Tt - 1, Ts, 2)

    gamma, read, write = pl.pallas_call(
        _gamma_body,
        grid=(B,),
        in_specs=[pl.BlockSpec((1, Tt, Ts), lambda b: (b, 0, 0))],
        out_specs=[
            pl.BlockSpec((1, Tt, Ts), lambda b: (b, 0, 0)),
            pl.BlockSpec((1, Tt - 1, Ts), lambda b: (b, 0, 0)),
            pl.BlockSpec((1, Tt - 1, Ts), lambda b: (b, 0, 0)),
        ],
        out_shape=[
            jax.ShapeDtypeStruct((B, Tt, Ts), jnp.float32),
            jax.ShapeDtypeStruct((B, Tt - 1, Ts), jnp.float32),
            jax.ShapeDtypeStruct((B, Tt - 1, Ts), jnp.float32),
        ],
        interpret=interpret,
    )(scores)

    return controls, gamma, read, write


@jax.jit
def kernel(observations, scores, gate_w, gate_b):
    return _run(observations, scores, gate_w, gate_b)


# speedup vs baseline: 6.9120x; 6.9120x over previous
import jax
import jax.numpy as jnp
from jax.experimental import pallas as pl
from jax.experimental.pallas import tpu as pltpu

_PENALTY = 0.1


def _controls_body(w_ref, b_ref, obs_ref, s_ref, sm_ref):
    blk = obs_ref.shape[1]
    ts = obs_ref.shape[2]
    c = obs_ref.shape[3]
    n = blk * ts
    m = obs_ref[0].reshape(n, c)
    x = jax.lax.dot_general(
        w_ref[...], m, (((1,), (1,)), ((), ())),
        preferred_element_type=jnp.float32,
    ) + b_ref[0, 0]
    s = jax.nn.log_sigmoid(x)
    s_ref[0, 0] = s
    sm_ref[0, 0] = s - x


def _gamma_body(scores_ref, gamma_ref, read_ref, write_ref):
    sc = scores_ref[0]
    Tt, Ts = sc.shape
    lane_i = jax.lax.broadcasted_iota(jnp.int32, (Tt, Ts), 1)
    lane_f = lane_i.astype(jnp.float32)
    scp = sc - _PENALTY * (lane_f / Ts)
    m = jnp.max(scp, axis=1, keepdims=True)
    cand = jnp.where(scp == m, lane_i, Ts)
    bc = jnp.min(cand, axis=1, keepdims=True)
    sub_i = jax.lax.broadcasted_iota(jnp.int32, (Tt, 1), 0)
    k = 1
    while k < Tt:
        shifted = pltpu.roll(bc, k, axis=0)
        bc = jnp.maximum(bc, jnp.where(sub_i >= k, shifted, -1))
        k *= 2
    gamma = (lane_i >= bc).astype(jnp.float32)
    gamma_ref[0] = gamma
    write_ref[0] = gamma[1:, :]
    read_ref[0] = 1.0 - gamma[1:, :]


def _run(observations, scores, gate_w, gate_b, interpret=False):
    B, Tt, Ts, C = observations.shape
    Tblk = 32
    nT = Tt // Tblk
    s_arr, sm_arr = pl.pallas_call(
        _controls_body,
        grid=(B, nT),
        in_specs=[
            pl.BlockSpec((1, C), lambda b, t: (0, 0)),
            pl.BlockSpec((1, 1), lambda b, t: (0, 0)),
            pl.BlockSpec((1, Tblk, Ts, C), lambda b, t: (b, t, 0, 0)),
        ],
        out_specs=[
            pl.BlockSpec((1, 1, 1, Tblk * Ts), lambda b, t: (b, t, 0, 0)),
            pl.BlockSpec((1, 1, 1, Tblk * Ts), lambda b, t: (b, t, 0, 0)),
        ],
        out_shape=[
            jax.ShapeDtypeStruct((B, nT, 1, Tblk * Ts), jnp.float32),
            jax.ShapeDtypeStruct((B, nT, 1, Tblk * Ts), jnp.float32),
        ],
        compiler_params=pltpu.CompilerParams(
            dimension_semantics=("parallel", "parallel"),
        ),
        interpret=interpret,
    )(gate_w, gate_b.reshape(1, 1), observations)
    s_arr = s_arr.reshape(B, Tt, Ts)
    sm_arr = sm_arr.reshape(B, Tt, Ts)
    controls = jnp.stack([s_arr, sm_arr], axis=-1)[:, :-1]

    gamma, read, write = pl.pallas_call(
        _gamma_body,
        grid=(B,),
        in_specs=[pl.BlockSpec((1, Tt, Ts), lambda b: (b, 0, 0))],
        out_specs=[
            pl.BlockSpec((1, Tt, Ts), lambda b: (b, 0, 0)),
            pl.BlockSpec((1, Tt - 1, Ts), lambda b: (b, 0, 0)),
            pl.BlockSpec((1, Tt - 1, Ts), lambda b: (b, 0, 0)),
        ],
        out_shape=[
            jax.ShapeDtypeStruct((B, Tt, Ts), jnp.float32),
            jax.ShapeDtypeStruct((B, Tt - 1, Ts), jnp.float32),
            jax.ShapeDtypeStruct((B, Tt - 1, Ts), jnp.float32),
        ],
        interpret=interpret,
    )(scores)

    return controls, gamma, read, write


@jax.jit
def kernel(observations, scores, gate_w, gate_b):
    return _run(observations, scores, gate_w, gate_b)


# speedup vs baseline: 8.0261x; 1.1612x over previous
import jax
import jax.numpy as jnp
from jax.experimental import pallas as pl
from jax.experimental.pallas import tpu as pltpu

_PENALTY = 0.1


def _controls_body(w_ref, b_ref, obs_ref, s_ref, sm_ref):
    blk = obs_ref.shape[1]
    ts = obs_ref.shape[2]
    c = obs_ref.shape[3]
    n = blk * ts
    m = obs_ref[0].reshape(n, c)
    x = jax.lax.dot_general(
        w_ref[...], m, (((1,), (1,)), ((), ())),
        preferred_element_type=jnp.float32,
    ) + b_ref[0, 0]
    s = jax.nn.log_sigmoid(x)
    s_ref[0, 0] = s
    sm_ref[0, 0] = s - x


def _gamma_body(scores_ref, gamma_ref, read_ref, write_ref):
    sc = scores_ref[0]
    Tt, Ts = sc.shape
    lane_i = jax.lax.broadcasted_iota(jnp.int32, (Tt, Ts), 1)
    lane_f = lane_i.astype(jnp.float32)
    scp = sc - _PENALTY * (lane_f / Ts)
    m = jnp.max(scp, axis=1, keepdims=True)
    cand = jnp.where(scp == m, lane_i, Ts)
    bc = jnp.min(cand, axis=1, keepdims=True)
    sub_i = jax.lax.broadcasted_iota(jnp.int32, (Tt, 1), 0)
    k = 1
    while k < Tt:
        shifted = pltpu.roll(bc, k, axis=0)
        bc = jnp.maximum(bc, jnp.where(sub_i >= k, shifted, -1))
        k *= 2
    gamma = (lane_i >= bc).astype(jnp.float32)
    gamma_ref[0] = gamma
    write_ref[0] = gamma[1:, :]
    read_ref[0] = 1.0 - gamma[1:, :]


def _run(observations, scores, gate_w, gate_b, interpret=False):
    B, Tt, Ts, C = observations.shape
    Tblk = 64
    nT = Tt // Tblk
    s_arr, sm_arr = pl.pallas_call(
        _controls_body,
        grid=(B, nT),
        in_specs=[
            pl.BlockSpec((1, C), lambda b, t: (0, 0)),
            pl.BlockSpec((1, 1), lambda b, t: (0, 0)),
            pl.BlockSpec((1, Tblk, Ts, C), lambda b, t: (b, t, 0, 0)),
        ],
        out_specs=[
            pl.BlockSpec((1, 1, 1, Tblk * Ts), lambda b, t: (b, t, 0, 0)),
            pl.BlockSpec((1, 1, 1, Tblk * Ts), lambda b, t: (b, t, 0, 0)),
        ],
        out_shape=[
            jax.ShapeDtypeStruct((B, nT, 1, Tblk * Ts), jnp.float32),
            jax.ShapeDtypeStruct((B, nT, 1, Tblk * Ts), jnp.float32),
        ],
        compiler_params=pltpu.CompilerParams(
            dimension_semantics=("parallel", "parallel"),
        ),
        interpret=interpret,
    )(gate_w, gate_b.reshape(1, 1), observations)
    s_arr = s_arr.reshape(B, Tt, Ts)
    sm_arr = sm_arr.reshape(B, Tt, Ts)
    controls = jnp.stack([s_arr, sm_arr], axis=-1)[:, :-1]

    gamma, read, write = pl.pallas_call(
        _gamma_body,
        grid=(B,),
        in_specs=[pl.BlockSpec((1, Tt, Ts), lambda b: (b, 0, 0))],
        out_specs=[
            pl.BlockSpec((1, Tt, Ts), lambda b: (b, 0, 0)),
            pl.BlockSpec((1, Tt - 1, Ts), lambda b: (b, 0, 0)),
            pl.BlockSpec((1, Tt - 1, Ts), lambda b: (b, 0, 0)),
        ],
        out_shape=[
            jax.ShapeDtypeStruct((B, Tt, Ts), jnp.float32),
            jax.ShapeDtypeStruct((B, Tt - 1, Ts), jnp.float32),
            jax.ShapeDtypeStruct((B, Tt - 1, Ts), jnp.float32),
        ],
        interpret=interpret,
    )(scores)

    return controls, gamma, read, write


@jax.jit
def kernel(observations, scores, gate_w, gate_b):
    return _run(observations, scores, gate_w, gate_b)
